# baseline (speedup 1.0000x reference)
import functools

import jax
import jax.numpy as jnp
from jax import lax
from jax.experimental import pallas as pl
from jax.experimental.pallas import tpu as pltpu
from jax.experimental.pallas import tpu_sc as plsc

VOCAB = 100000
HIDDEN = 128
BATCH = 4096
SEQ = 50

NC = 2
NS = 16
NW = NC * NS

P = 4
BP = BATCH // P
ROWS_W = BP // NW
NBUF = 8
NG = ROWS_W // NBUF


def _embed_body(idx_hbm, table_hbm, out_hbm, idx_v, rows_v, sems):
    c = lax.axis_index("c")
    s = lax.axis_index("s")
    wid = s * NC + c
    base = wid * ROWS_W

    pltpu.sync_copy(idx_hbm.at[pl.ds(base, ROWS_W)], idx_v)

    for b in range(NBUF):
        pltpu.async_copy(table_hbm.at[idx_v.at[b]], rows_v.at[b], sems.at[b])

    def group(g, carry):
        for b in range(NBUF):
            j = g * NBUF + b
            pltpu.make_async_copy(
                table_hbm.at[idx_v.at[j]], rows_v.at[b], sems.at[b]
            ).wait()
            pltpu.sync_copy(rows_v.at[b], out_hbm.at[base + j])
            pltpu.async_copy(
                table_hbm.at[idx_v.at[j + NBUF]], rows_v.at[b], sems.at[b]
            )
        return carry

    lax.fori_loop(0, NG - 1, group, 0)

    for b in range(NBUF):
        j = (NG - 1) * NBUF + b
        pltpu.make_async_copy(
            table_hbm.at[idx_v.at[j]], rows_v.at[b], sems.at[b]
        ).wait()
        pltpu.sync_copy(rows_v.at[b], out_hbm.at[base + j])


def _embed(idx, table):
    mesh = plsc.VectorSubcoreMesh(core_axis_name="c", subcore_axis_name="s")
    return pl.kernel(
        _embed_body,
        mesh=mesh,
        out_type=jax.ShapeDtypeStruct((BP, SEQ, HIDDEN), jnp.float32),
        scratch_types=[
            pltpu.VMEM((ROWS_W, SEQ), jnp.int32),
            pltpu.VMEM((NBUF, SEQ, HIDDEN), jnp.float32),
            pltpu.SemaphoreType.DMA((NBUF,)),
        ],
    )(idx, table)


def kernel(inputs, embedding):
    idx = inputs.astype(jnp.int32)
    parts = [
        _embed(lax.slice_in_dim(idx, k * BP, (k + 1) * BP, axis=0), embedding)
        for k in range(P)
    ]
    return jnp.concatenate(parts, axis=0)


# speedup vs baseline: 1.0162x; 1.0162x over previous
import functools

import jax
import jax.numpy as jnp
from jax import lax
from jax.experimental import pallas as pl
from jax.experimental.pallas import tpu as pltpu
from jax.experimental.pallas import tpu_sc as plsc

VOCAB = 100000
HIDDEN = 128
BATCH = 4096
SEQ = 50

NC = 2
NS = 16
NW = NC * NS

P = 4
BP = BATCH // P
ROWS_W = BP // NW
NBUF = 8
NG = ROWS_W // NBUF


def _embed_body(idx_hbm, table_hbm, out_hbm, idx_v, rows_v, sems):
    c = lax.axis_index("c")
    s = lax.axis_index("s")
    wid = s * NC + c
    base = wid * ROWS_W

    pltpu.sync_copy(idx_hbm.at[pl.ds(base, ROWS_W)], idx_v)

    for b in range(NBUF):
        pltpu.async_copy(table_hbm.at[idx_v.at[b]], rows_v.at[b], sems.at[b])

    def group(g, carry):
        for b in range(NBUF):
            j = g * NBUF + b
            pltpu.make_async_copy(
                table_hbm.at[idx_v.at[j]], rows_v.at[b], sems.at[b]
            ).wait()
            pltpu.sync_copy(rows_v.at[b], out_hbm.at[base + j])
            pltpu.async_copy(
                table_hbm.at[idx_v.at[j + NBUF]], rows_v.at[b], sems.at[b]
            )
        return carry

    lax.fori_loop(0, NG - 1, group, 0)

    for b in range(NBUF):
        j = (NG - 1) * NBUF + b
        pltpu.make_async_copy(
            table_hbm.at[idx_v.at[j]], rows_v.at[b], sems.at[b]
        ).wait()
        pltpu.sync_copy(rows_v.at[b], out_hbm.at[base + j])


def _embed(idx, table):
    mesh = plsc.VectorSubcoreMesh(core_axis_name="c", subcore_axis_name="s")
    return pl.kernel(
        _embed_body,
        mesh=mesh,
        out_type=jax.ShapeDtypeStruct((BP, SEQ, HIDDEN), jnp.float32),
        scratch_types=[
            pltpu.VMEM((ROWS_W, SEQ), jnp.int32),
            pltpu.VMEM((NBUF, SEQ, HIDDEN), jnp.float32),
            pltpu.SemaphoreType.DMA((NBUF,)),
        ],
    )(idx, table)


def kernel(inputs, embedding):
    idx = inputs.astype(jnp.int32)
    out = jnp.zeros((BATCH, SEQ, HIDDEN), jnp.float32)
    for k in range(P):
        part = _embed(
            lax.slice_in_dim(idx, k * BP, (k + 1) * BP, axis=0), embedding
        )
        out = lax.dynamic_update_slice(out, part, (k * BP, 0, 0))
    return out


# speedup vs baseline: 3.2162x; 3.1650x over previous
import functools

import jax
import jax.numpy as jnp
from jax import lax
from jax.experimental import pallas as pl
from jax.experimental.pallas import tpu as pltpu
from jax.experimental.pallas import tpu_sc as plsc

VOCAB = 100000
HIDDEN = 128
BATCH = 4096
SEQ = 50

NC = 2
NS = 16
NW = NC * NS
COLS_W = BATCH // NW
NBUF = 5
NG = SEQ // NBUF


def _embed_body(idx_hbm, table_hbm, out_hbm, idx_v, rows_v, sems):
    c = lax.axis_index("c")
    s = lax.axis_index("s")
    wid = s * NC + c
    col = wid * COLS_W

    pltpu.sync_copy(idx_hbm.at[:, pl.ds(col, COLS_W)], idx_v)

    for b in range(NBUF):
        pltpu.async_copy(table_hbm.at[idx_v.at[b]], rows_v.at[b], sems.at[b])

    def group(g, carry):
        for b in range(NBUF):
            j = g * NBUF + b
            pltpu.make_async_copy(
                table_hbm.at[idx_v.at[j]], rows_v.at[b], sems.at[b]
            ).wait()
            pltpu.sync_copy(rows_v.at[b], out_hbm.at[j, pl.ds(col, COLS_W)])
            pltpu.async_copy(
                table_hbm.at[idx_v.at[j + NBUF]], rows_v.at[b], sems.at[b]
            )
        return carry

    lax.fori_loop(0, NG - 1, group, 0)

    for b in range(NBUF):
        j = (NG - 1) * NBUF + b
        pltpu.make_async_copy(
            table_hbm.at[idx_v.at[j]], rows_v.at[b], sems.at[b]
        ).wait()
        pltpu.sync_copy(rows_v.at[b], out_hbm.at[j, pl.ds(col, COLS_W)])


@functools.partial(jax.jit, static_argnums=())
def _embed(idx_t, table):
    mesh = plsc.VectorSubcoreMesh(core_axis_name="c", subcore_axis_name="s")
    return pl.kernel(
        _embed_body,
        mesh=mesh,
        out_type=jax.ShapeDtypeStruct((SEQ, BATCH, HIDDEN), jnp.float32),
        scratch_types=[
            pltpu.VMEM((SEQ, COLS_W), jnp.int32),
            pltpu.VMEM((NBUF, COLS_W, HIDDEN), jnp.float32),
            pltpu.SemaphoreType.DMA((NBUF,)),
        ],
    )(idx_t, table)


def kernel(inputs, embedding):
    idx_t = inputs.astype(jnp.int32).T
    out = _embed(idx_t, embedding)
    return out.transpose(1, 0, 2)


# speedup vs baseline: 3.2280x; 1.0037x over previous
import functools

import jax
import jax.numpy as jnp
from jax import lax
from jax.experimental import pallas as pl
from jax.experimental.pallas import tpu as pltpu
from jax.experimental.pallas import tpu_sc as plsc

VOCAB = 100000
HIDDEN = 128
BATCH = 4096
SEQ = 50

NC = 2
NS = 16
NW = NC * NS
COLS_W = BATCH // NW
NBUF = 5
NG = SEQ // NBUF


def _embed_body(idx_hbm, table_hbm, out_hbm, idx_v, rows_v, gsems, wsems):
    c = lax.axis_index("c")
    s = lax.axis_index("s")
    wid = s * NC + c
    col = wid * COLS_W

    def gather(j, b):
        pltpu.async_copy(table_hbm.at[idx_v.at[j]], rows_v.at[b], gsems.at[b])

    def wait_gather(j, b):
        pltpu.make_async_copy(
            table_hbm.at[idx_v.at[j]], rows_v.at[b], gsems.at[b]
        ).wait()

    def write(j, b):
        pltpu.async_copy(
            rows_v.at[b], out_hbm.at[j, pl.ds(col, COLS_W)], wsems.at[b]
        )

    def wait_write(j, b):
        pltpu.make_async_copy(
            rows_v.at[b], out_hbm.at[j, pl.ds(col, COLS_W)], wsems.at[b]
        ).wait()

    pltpu.sync_copy(idx_hbm.at[:, pl.ds(col, COLS_W)], idx_v)

    for b in range(NBUF):
        gather(b, b)
    wait_gather(0, 0)
    write(0, 0)

    def group(g, carry):
        for i in range(NBUF):
            j = g * NBUF + i + 1
            b = j % NBUF
            bp = (j - 1) % NBUF
            wait_write(j - 1, bp)
            gather(j - 1 + NBUF, bp)
            wait_gather(j, b)
            write(j, b)
        return carry

    lax.fori_loop(0, (SEQ - NBUF) // NBUF, group, 0)

    for j in range(SEQ - NBUF + 1, SEQ):
        b = j % NBUF
        wait_gather(j, b)
        write(j, b)

    for j in range(SEQ - NBUF, SEQ):
        wait_write(j, j % NBUF)


@functools.partial(jax.jit, static_argnums=())
def _embed(idx_t, table):
    mesh = plsc.VectorSubcoreMesh(core_axis_name="c", subcore_axis_name="s")
    return pl.kernel(
        _embed_body,
        mesh=mesh,
        out_type=jax.ShapeDtypeStruct((SEQ, BATCH, HIDDEN), jnp.float32),
        scratch_types=[
            pltpu.VMEM((SEQ, COLS_W), jnp.int32),
            pltpu.VMEM((NBUF, COLS_W, HIDDEN), jnp.float32),
            pltpu.SemaphoreType.DMA((NBUF,)),
            pltpu.SemaphoreType.DMA((NBUF,)),
        ],
    )(idx_t, table)


def kernel(inputs, embedding):
    idx_t = inputs.astype(jnp.int32).T
    out = _embed(idx_t, embedding)
    return out.transpose(1, 0, 2)
